# baseline (speedup 1.0000x reference)
import functools

import jax
import jax.numpy as jnp
from jax import lax
from jax.experimental import pallas as pl
from jax.experimental.pallas import tpu as pltpu
from jax.experimental.pallas import tpu_sc as plsc

_H = 128
_NC = 2
_NS = 16
_NW = _NC * _NS


def _edge_proj(edge_attr, a_t, c):
    e = edge_attr.shape[0]
    be = 3200
    grid = e // be

    def body(ea_ref, w_ref, b_ref, o_ref):
        o_ref[...] = jnp.dot(ea_ref[...], w_ref[...],
                             preferred_element_type=jnp.float32) + b_ref[...]

    return pl.pallas_call(
        body,
        grid=(grid,),
        in_specs=[
            pl.BlockSpec((be, 16), lambda i: (i, 0)),
            pl.BlockSpec((16, _H), lambda i: (0, 0)),
            pl.BlockSpec((1, _H), lambda i: (0, 0)),
        ],
        out_specs=pl.BlockSpec((be, _H), lambda i: (i, 0)),
        out_shape=jax.ShapeDtypeStruct((e, _H), jnp.float32),
    )(edge_attr, a_t, c)


def _node_proj(x, w_t, b):
    n = x.shape[0]
    bn = 2000
    grid = n // bn

    def body(x_ref, w_ref, b_ref, k_ref, q_ref, v_ref, s_ref):
        acc = jnp.dot(x_ref[...], w_ref[...],
                      preferred_element_type=jnp.float32) + b_ref[...]
        k_ref[...] = acc[:, 0 * _H:1 * _H]
        q_ref[...] = acc[:, 1 * _H:2 * _H]
        v_ref[...] = acc[:, 2 * _H:3 * _H]
        s_ref[...] = acc[:, 3 * _H:4 * _H]

    return pl.pallas_call(
        body,
        grid=(grid,),
        in_specs=[
            pl.BlockSpec((bn, _H), lambda i: (i, 0)),
            pl.BlockSpec((_H, 4 * _H), lambda i: (0, 0)),
            pl.BlockSpec((1, 4 * _H), lambda i: (0, 0)),
        ],
        out_specs=[pl.BlockSpec((bn, _H), lambda i: (i, 0))] * 4,
        out_shape=[jax.ShapeDtypeStruct((n, _H), jnp.float32)] * 4,
    )(x, w_t, b)


def _ln_rows(h, g_ref, b_ref):
    mu = jnp.mean(h, axis=1, keepdims=True)
    zc = h - mu
    var = jnp.mean(zc * zc, axis=1, keepdims=True)
    return zc * lax.rsqrt(var + 1e-5) * g_ref[...] + b_ref[...]


def _combine_ln_proj(agg2, skip, g, b, w_t, bias):
    n = skip.shape[0]
    bn = 2000
    grid = n // bn

    def body(a_ref, s_ref, g_ref, b_ref, w_ref, bb_ref,
             h_ref, k_ref, q_ref, v_ref, sk_ref):
        hn = _ln_rows(a_ref[0] + a_ref[1] + s_ref[...], g_ref, b_ref)
        h_ref[...] = hn
        acc = jnp.dot(hn, w_ref[...],
                      preferred_element_type=jnp.float32) + bb_ref[...]
        k_ref[...] = acc[:, 0 * _H:1 * _H]
        q_ref[...] = acc[:, 1 * _H:2 * _H]
        v_ref[...] = acc[:, 2 * _H:3 * _H]
        sk_ref[...] = acc[:, 3 * _H:4 * _H]

    return pl.pallas_call(
        body,
        grid=(grid,),
        in_specs=[
            pl.BlockSpec((2, bn, _H), lambda i: (0, i, 0)),
            pl.BlockSpec((bn, _H), lambda i: (i, 0)),
            pl.BlockSpec((1, _H), lambda i: (0, 0)),
            pl.BlockSpec((1, _H), lambda i: (0, 0)),
            pl.BlockSpec((_H, 4 * _H), lambda i: (0, 0)),
            pl.BlockSpec((1, 4 * _H), lambda i: (0, 0)),
        ],
        out_specs=[pl.BlockSpec((bn, _H), lambda i: (i, 0))] * 5,
        out_shape=[jax.ShapeDtypeStruct((n, _H), jnp.float32)] * 5,
    )(agg2, skip, g, b, w_t, bias)


def _combine_ln_final(agg2, skip, g, b, h1, h2, w_t, bias):
    n = skip.shape[0]
    bn = 2000
    grid = n // bn

    def body(a_ref, s_ref, g_ref, b_ref, h1_ref, h2_ref, w_ref, bb_ref, o_ref):
        hn = _ln_rows(a_ref[0] + a_ref[1] + s_ref[...], g_ref, b_ref)
        o_ref[...] = (
            jnp.dot(h1_ref[...], w_ref[0 * _H:1 * _H, :],
                    preferred_element_type=jnp.float32)
            + jnp.dot(h2_ref[...], w_ref[1 * _H:2 * _H, :],
                      preferred_element_type=jnp.float32)
            + jnp.dot(hn, w_ref[2 * _H:3 * _H, :],
                      preferred_element_type=jnp.float32)
            + bb_ref[...])

    return pl.pallas_call(
        body,
        grid=(grid,),
        in_specs=[
            pl.BlockSpec((2, bn, _H), lambda i: (0, i, 0)),
            pl.BlockSpec((bn, _H), lambda i: (i, 0)),
            pl.BlockSpec((1, _H), lambda i: (0, 0)),
            pl.BlockSpec((1, _H), lambda i: (0, 0)),
            pl.BlockSpec((bn, _H), lambda i: (i, 0)),
            pl.BlockSpec((bn, _H), lambda i: (i, 0)),
            pl.BlockSpec((3 * _H, _H), lambda i: (0, 0)),
            pl.BlockSpec((1, _H), lambda i: (0, 0)),
        ],
        out_specs=pl.BlockSpec((bn, _H), lambda i: (i, 0)),
        out_shape=jax.ShapeDtypeStruct((n, _H), jnp.float32),
    )(agg2, skip, g, b, h1, h2, w_t, bias)


def _sc_gate_agg(k, q, v, el, src, dst):
    n = k.shape[0]
    e = src.shape[0]
    chunk = 40
    e_tile = e // _NW
    n_chunks = e_tile // chunk
    n_pad = ((n + _NS * chunk - 1) // (_NS * chunk)) * (_NS * chunk)
    rows = n_pad // _NS

    mesh = plsc.VectorSubcoreMesh(core_axis_name="c", subcore_axis_name="s")

    vm = pltpu.VMEM

    @functools.partial(
        pl.kernel,
        mesh=mesh,
        out_type=jax.ShapeDtypeStruct((_NC, n_pad, _H), jnp.float32),
        scratch_types=[
            pltpu.VMEM_SHARED((n_pad, _H), jnp.float32),
            [pltpu.SemaphoreType.DMA] * 2,
            [pltpu.SemaphoreType.DMA] * 2,
        ],
    )
    def body(k_hbm, q_hbm, v_hbm, el_hbm, src_hbm, dst_hbm, out_hbm,
             agg, gsem, isem):
        pl.run_scoped(
            functools.partial(_sc_body_scoped, k_hbm, q_hbm, v_hbm, el_hbm,
                              src_hbm, dst_hbm, out_hbm, agg, gsem, isem,
                              chunk, e_tile, n_chunks, rows),
            dstv=[vm((chunk,), jnp.int32)] * 2,
            srcv=[vm((chunk,), jnp.int32)] * 2,
            kdv=[vm((chunk, _H), jnp.float32)] * 2,
            qsv=[vm((chunk, _H), jnp.float32)] * 2,
            vsv=[vm((chunk, _H), jnp.float32)] * 2,
            elv=[vm((chunk, _H), jnp.float32)] * 2,
        )

    return body(k, q, v, el, src, dst)


def _sc_body_scoped(k_hbm, q_hbm, v_hbm, el_hbm, src_hbm, dst_hbm, out_hbm,
                    agg, gsem, isem, chunk, e_tile, n_chunks, rows,
                    dstv, srcv, kdv, qsv, vsv, elv):
        cid = lax.axis_index("c")
        sid = lax.axis_index("s")
        wid = sid * _NC + cid
        tile_base = wid * e_tile

        zero16 = jnp.zeros((16,), jnp.float32)

        def zrow(i, _):
            for hh in range(_H // 16):
                elv[0][i, pl.ds(hh * 16, 16)] = zero16
            return 0

        lax.fori_loop(0, chunk, zrow, 0)

        def zslice(j, _):
            pltpu.sync_copy(elv[0],
                            agg.at[pl.ds(sid * rows + j * chunk, chunk)])
            return 0

        lax.fori_loop(0, rows // chunk, zslice, 0)
        plsc.subcore_barrier()

        def idx_load(g2, b):
            base = tile_base + g2 * chunk
            pltpu.async_copy(dst_hbm.at[pl.ds(base, chunk)], dstv[b], isem[b])
            pltpu.async_copy(src_hbm.at[pl.ds(base, chunk)], srcv[b], isem[b])

        def idx_wait(g2, b):
            base = tile_base + g2 * chunk
            pltpu.make_async_copy(
                dst_hbm.at[pl.ds(base, chunk)], dstv[b], isem[b]).wait()
            pltpu.make_async_copy(
                src_hbm.at[pl.ds(base, chunk)], srcv[b], isem[b]).wait()

        def gathers(g2, b):
            base = tile_base + g2 * chunk
            pltpu.async_copy(k_hbm.at[dstv[b]], kdv[b], gsem[b])
            pltpu.async_copy(q_hbm.at[srcv[b]], qsv[b], gsem[b])
            pltpu.async_copy(v_hbm.at[srcv[b]], vsv[b], gsem[b])
            pltpu.async_copy(el_hbm.at[pl.ds(base, chunk)], elv[b], gsem[b])

        idx_load(0, 0)
        idx_wait(0, 0)
        gathers(0, 0)
        idx_load(1, 1)

        def pair(i, _):
            for b in (0, 1):
                g = 2 * i + b
                ob = 1 - b
                @pl.when(g + 1 < n_chunks)
                def _():
                    idx_wait(g + 1, ob)
                    gathers(g + 1, ob)

                pltpu.make_async_copy(k_hbm.at[dstv[b]], kdv[b], gsem[b]).wait()
                pltpu.make_async_copy(q_hbm.at[srcv[b]], qsv[b], gsem[b]).wait()
                pltpu.make_async_copy(v_hbm.at[srcv[b]], vsv[b], gsem[b]).wait()
                pltpu.make_async_copy(
                    el_hbm.at[pl.ds(tile_base, chunk)], elv[b], gsem[b]).wait()

                def edge(j, _):
                    for hh in range(_H // 16):
                        sl = pl.ds(hh * 16, 16)
                        z = kdv[b][j, sl] + elv[b][j, sl] + qsv[b][j, sl]
                        gate = 1.0 / (1.0 + jnp.exp(-z))
                        elv[b][j, sl] = gate * vsv[b][j, sl]
                    return 0

                lax.fori_loop(0, chunk, edge, 0)
                pltpu.sync_copy(elv[b], agg.at[dstv[b]], add=True)

                @pl.when(g + 2 < n_chunks)
                def _():
                    idx_load(g + 2, b)

            return 0

        lax.fori_loop(0, n_chunks // 2, pair, 0)
        plsc.subcore_barrier()
        pltpu.sync_copy(agg.at[pl.ds(sid * rows, rows)],
                        out_hbm.at[cid, pl.ds(sid * rows, rows)])


def kernel(node_features, edge_index, edge_attr, params):
    src = edge_index[0].astype(jnp.int32)
    dst = edge_index[1].astype(jnp.int32)
    layers = params['layers']

    els = [
        _edge_proj(edge_attr,
                   (p['We'] @ params['W_edge']).T,
                   (p['We'] @ params['b_edge'] + p['be'])[None, :])
        for p in layers
    ]

    def cat_w(p):
        w = jnp.concatenate([p['Wk'], p['Wq'], p['Wv'], p['Wskip']], axis=0)
        b = jnp.concatenate([p['bk'], p['bq'], p['bv'], p['bskip']], axis=0)
        return w, b

    w0, b0 = cat_w(layers[0])
    k, q, v, s = _node_proj(node_features, (w0 @ params['W_node']).T,
                            (w0 @ params['b_node'] + b0)[None, :])

    hs = []
    for li in range(len(layers)):
        p = layers[li]
        agg2 = _sc_gate_agg(k, q, v, els[li], src, dst)
        ln_g, ln_b = p['ln_g'][None, :], p['ln_b'][None, :]
        if li + 1 < len(layers):
            wn, bn_ = cat_w(layers[li + 1])
            h, k, q, v, s = _combine_ln_proj(agg2, s, ln_g, ln_b,
                                             wn.T, bn_[None, :])
            hs.append(h)
        else:
            return _combine_ln_final(agg2, s, ln_g, ln_b, hs[0], hs[1],
                                     params['W_hidden'].T,
                                     params['b_hidden'][None, :])


# speedup vs baseline: 1.0635x; 1.0635x over previous
import functools

import jax
import jax.numpy as jnp
from jax import lax
from jax.experimental import pallas as pl
from jax.experimental.pallas import tpu as pltpu
from jax.experimental.pallas import tpu_sc as plsc

_H = 128
_NC = 2
_NS = 16
_NW = _NC * _NS


def _edge_proj(edge_attr, a_t, c):
    e = edge_attr.shape[0]
    be = 3200
    grid = e // be

    def body(ea_ref, w_ref, b_ref, o_ref):
        o_ref[...] = jnp.dot(ea_ref[...], w_ref[...],
                             preferred_element_type=jnp.float32) + b_ref[...]

    return pl.pallas_call(
        body,
        grid=(grid,),
        in_specs=[
            pl.BlockSpec((be, 16), lambda i: (i, 0)),
            pl.BlockSpec((16, _H), lambda i: (0, 0)),
            pl.BlockSpec((1, _H), lambda i: (0, 0)),
        ],
        out_specs=pl.BlockSpec((be, _H), lambda i: (i, 0)),
        out_shape=jax.ShapeDtypeStruct((e, _H), jnp.float32),
    )(edge_attr, a_t, c)


def _node_proj(x, w_t, b):
    n = x.shape[0]
    bn = 2000
    grid = n // bn

    def body(x_ref, w_ref, b_ref, k_ref, q_ref, v_ref, s_ref):
        acc = jnp.dot(x_ref[...], w_ref[...],
                      preferred_element_type=jnp.float32) + b_ref[...]
        k_ref[...] = acc[:, 0 * _H:1 * _H]
        q_ref[...] = acc[:, 1 * _H:2 * _H]
        v_ref[...] = acc[:, 2 * _H:3 * _H]
        s_ref[...] = acc[:, 3 * _H:4 * _H]

    return pl.pallas_call(
        body,
        grid=(grid,),
        in_specs=[
            pl.BlockSpec((bn, _H), lambda i: (i, 0)),
            pl.BlockSpec((_H, 4 * _H), lambda i: (0, 0)),
            pl.BlockSpec((1, 4 * _H), lambda i: (0, 0)),
        ],
        out_specs=[pl.BlockSpec((bn, _H), lambda i: (i, 0))] * 4,
        out_shape=[jax.ShapeDtypeStruct((n, _H), jnp.float32)] * 4,
    )(x, w_t, b)


def _ln_rows(h, g_ref, b_ref):
    mu = jnp.mean(h, axis=1, keepdims=True)
    zc = h - mu
    var = jnp.mean(zc * zc, axis=1, keepdims=True)
    return zc * lax.rsqrt(var + 1e-5) * g_ref[...] + b_ref[...]


def _combine_ln_proj(agg2, skip, g, b, w_t, bias):
    n = skip.shape[0]
    bn = 2000
    grid = n // bn

    def body(a_ref, s_ref, g_ref, b_ref, w_ref, bb_ref,
             h_ref, k_ref, q_ref, v_ref, sk_ref):
        hn = _ln_rows(a_ref[0] + a_ref[1] + s_ref[...], g_ref, b_ref)
        h_ref[...] = hn
        acc = jnp.dot(hn, w_ref[...],
                      preferred_element_type=jnp.float32) + bb_ref[...]
        k_ref[...] = acc[:, 0 * _H:1 * _H]
        q_ref[...] = acc[:, 1 * _H:2 * _H]
        v_ref[...] = acc[:, 2 * _H:3 * _H]
        sk_ref[...] = acc[:, 3 * _H:4 * _H]

    return pl.pallas_call(
        body,
        grid=(grid,),
        in_specs=[
            pl.BlockSpec((2, bn, _H), lambda i: (0, i, 0)),
            pl.BlockSpec((bn, _H), lambda i: (i, 0)),
            pl.BlockSpec((1, _H), lambda i: (0, 0)),
            pl.BlockSpec((1, _H), lambda i: (0, 0)),
            pl.BlockSpec((_H, 4 * _H), lambda i: (0, 0)),
            pl.BlockSpec((1, 4 * _H), lambda i: (0, 0)),
        ],
        out_specs=[pl.BlockSpec((bn, _H), lambda i: (i, 0))] * 5,
        out_shape=[jax.ShapeDtypeStruct((n, _H), jnp.float32)] * 5,
    )(agg2, skip, g, b, w_t, bias)


def _combine_ln_final(agg2, skip, g, b, h1, h2, w_t, bias):
    n = skip.shape[0]
    bn = 2000
    grid = n // bn

    def body(a_ref, s_ref, g_ref, b_ref, h1_ref, h2_ref, w_ref, bb_ref, o_ref):
        hn = _ln_rows(a_ref[0] + a_ref[1] + s_ref[...], g_ref, b_ref)
        o_ref[...] = (
            jnp.dot(h1_ref[...], w_ref[0 * _H:1 * _H, :],
                    preferred_element_type=jnp.float32)
            + jnp.dot(h2_ref[...], w_ref[1 * _H:2 * _H, :],
                      preferred_element_type=jnp.float32)
            + jnp.dot(hn, w_ref[2 * _H:3 * _H, :],
                      preferred_element_type=jnp.float32)
            + bb_ref[...])

    return pl.pallas_call(
        body,
        grid=(grid,),
        in_specs=[
            pl.BlockSpec((2, bn, _H), lambda i: (0, i, 0)),
            pl.BlockSpec((bn, _H), lambda i: (i, 0)),
            pl.BlockSpec((1, _H), lambda i: (0, 0)),
            pl.BlockSpec((1, _H), lambda i: (0, 0)),
            pl.BlockSpec((bn, _H), lambda i: (i, 0)),
            pl.BlockSpec((bn, _H), lambda i: (i, 0)),
            pl.BlockSpec((3 * _H, _H), lambda i: (0, 0)),
            pl.BlockSpec((1, _H), lambda i: (0, 0)),
        ],
        out_specs=pl.BlockSpec((bn, _H), lambda i: (i, 0)),
        out_shape=jax.ShapeDtypeStruct((n, _H), jnp.float32),
    )(agg2, skip, g, b, h1, h2, w_t, bias)


def _sc_gate_agg(k, q, v, el, src, dst):
    n = k.shape[0]
    e = src.shape[0]
    chunk = 40
    e_tile = e // _NW
    n_chunks = e_tile // chunk
    n_pad = ((n + _NS * chunk - 1) // (_NS * chunk)) * (_NS * chunk)
    rows = n_pad // _NS

    mesh = plsc.VectorSubcoreMesh(core_axis_name="c", subcore_axis_name="s")

    vm = pltpu.VMEM

    @functools.partial(
        pl.kernel,
        mesh=mesh,
        out_type=jax.ShapeDtypeStruct((_NC, n_pad, _H), jnp.float32),
        scratch_types=[
            pltpu.VMEM_SHARED((n_pad, _H), jnp.float32),
            [pltpu.SemaphoreType.DMA] * 2,
            [pltpu.SemaphoreType.DMA] * 2,
            [pltpu.SemaphoreType.DMA] * 2,
        ],
    )
    def body(k_hbm, q_hbm, v_hbm, el_hbm, src_hbm, dst_hbm, out_hbm,
             agg, gsem, isem, ssem):
        pl.run_scoped(
            functools.partial(_sc_body_scoped, k_hbm, q_hbm, v_hbm, el_hbm,
                              src_hbm, dst_hbm, out_hbm, agg, gsem, isem,
                              ssem, chunk, e_tile, n_chunks, rows),
            dstv=[vm((chunk,), jnp.int32)] * 4,
            srcv=[vm((chunk,), jnp.int32)] * 2,
            kdv=[vm((chunk, _H), jnp.float32)] * 2,
            qsv=[vm((chunk, _H), jnp.float32)] * 2,
            vsv=[vm((chunk, _H), jnp.float32)] * 2,
            elv=[vm((chunk, _H), jnp.float32)] * 2,
        )

    return body(k, q, v, el, src, dst)


def _sc_body_scoped(k_hbm, q_hbm, v_hbm, el_hbm, src_hbm, dst_hbm, out_hbm,
                    agg, gsem, isem, ssem, chunk, e_tile, n_chunks, rows,
                    dstv, srcv, kdv, qsv, vsv, elv):
        cid = lax.axis_index("c")
        sid = lax.axis_index("s")
        wid = sid * _NC + cid
        tile_base = wid * e_tile

        zero16 = jnp.zeros((16,), jnp.float32)

        def zrow(i, _):
            for hh in range(_H // 16):
                elv[0][i, pl.ds(hh * 16, 16)] = zero16
            return 0

        lax.fori_loop(0, chunk, zrow, 0)

        def zslice(j, _):
            pltpu.sync_copy(elv[0],
                            agg.at[pl.ds(sid * rows + j * chunk, chunk)])
            return 0

        lax.fori_loop(0, rows // chunk, zslice, 0)
        plsc.subcore_barrier()

        def idx_load(g2, d, b):
            base = tile_base + g2 * chunk
            pltpu.async_copy(dst_hbm.at[pl.ds(base, chunk)], dstv[d], isem[b])
            pltpu.async_copy(src_hbm.at[pl.ds(base, chunk)], srcv[b], isem[b])

        def idx_wait(g2, d, b):
            base = tile_base + g2 * chunk
            pltpu.make_async_copy(
                dst_hbm.at[pl.ds(base, chunk)], dstv[d], isem[b]).wait()
            pltpu.make_async_copy(
                src_hbm.at[pl.ds(base, chunk)], srcv[b], isem[b]).wait()

        def gathers(g2, d, b):
            base = tile_base + g2 * chunk
            pltpu.async_copy(k_hbm.at[dstv[d]], kdv[b], gsem[b])
            pltpu.async_copy(q_hbm.at[srcv[b]], qsv[b], gsem[b])
            pltpu.async_copy(v_hbm.at[srcv[b]], vsv[b], gsem[b])
            pltpu.async_copy(el_hbm.at[pl.ds(base, chunk)], elv[b], gsem[b])

        def phase(g, j, wait_prev, next_gather=True, load_ahead=True):
            b = j % 2
            d = j
            ob = 1 - b
            if next_gather:
                idx_wait(g + 1, (j + 1) % 4, ob)
                if wait_prev:
                    pltpu.make_async_copy(
                        kdv[ob], agg.at[dstv[(j + 3) % 4]], ssem[ob]).wait()
                gathers(g + 1, (j + 1) % 4, ob)

            pltpu.make_async_copy(k_hbm.at[dstv[d]], kdv[b], gsem[b]).wait()
            pltpu.make_async_copy(q_hbm.at[srcv[b]], qsv[b], gsem[b]).wait()
            pltpu.make_async_copy(v_hbm.at[srcv[b]], vsv[b], gsem[b]).wait()
            pltpu.make_async_copy(
                el_hbm.at[pl.ds(tile_base, chunk)], elv[b], gsem[b]).wait()

            def edge(jj, _):
                for hh in range(_H // 16):
                    sl = pl.ds(hh * 16, 16)
                    z = kdv[b][jj, sl] + elv[b][jj, sl] + qsv[b][jj, sl]
                    gate = 1.0 / (1.0 + jnp.exp(-z))
                    kdv[b][jj, sl] = gate * vsv[b][jj, sl]
                return 0

            lax.fori_loop(0, chunk, edge, 0)
            pltpu.async_copy(kdv[b], agg.at[dstv[d]], ssem[b], add=True)

            if load_ahead:
                idx_load(g + 2, (j + 2) % 4, b)

        idx_load(0, 0, 0)
        idx_wait(0, 0, 0)
        gathers(0, 0, 0)
        idx_load(1, 1, 1)
        phase(0, 0, wait_prev=False)
        phase(1, 1, wait_prev=True)
        phase(2, 2, wait_prev=True)
        phase(3, 3, wait_prev=True)

        def quad(i, _):
            for j in range(4):
                phase(4 * i + j, j, wait_prev=True)
            return 0

        lax.fori_loop(1, n_chunks // 4, quad, 0)

        phase(n_chunks - 2, 0, wait_prev=True, next_gather=True,
              load_ahead=False)
        phase(n_chunks - 1, 1, wait_prev=False, next_gather=False,
              load_ahead=False)
        pltpu.make_async_copy(kdv[0], agg.at[dstv[0]], ssem[0]).wait()
        pltpu.make_async_copy(kdv[1], agg.at[dstv[1]], ssem[1]).wait()
        plsc.subcore_barrier()
        pltpu.sync_copy(agg.at[pl.ds(sid * rows, rows)],
                        out_hbm.at[cid, pl.ds(sid * rows, rows)])


def kernel(node_features, edge_index, edge_attr, params):
    src = edge_index[0].astype(jnp.int32)
    dst = edge_index[1].astype(jnp.int32)
    layers = params['layers']

    els = [
        _edge_proj(edge_attr,
                   (p['We'] @ params['W_edge']).T,
                   (p['We'] @ params['b_edge'] + p['be'])[None, :])
        for p in layers
    ]

    def cat_w(p):
        w = jnp.concatenate([p['Wk'], p['Wq'], p['Wv'], p['Wskip']], axis=0)
        b = jnp.concatenate([p['bk'], p['bq'], p['bv'], p['bskip']], axis=0)
        return w, b

    w0, b0 = cat_w(layers[0])
    k, q, v, s = _node_proj(node_features, (w0 @ params['W_node']).T,
                            (w0 @ params['b_node'] + b0)[None, :])

    hs = []
    for li in range(len(layers)):
        p = layers[li]
        agg2 = _sc_gate_agg(k, q, v, els[li], src, dst)
        ln_g, ln_b = p['ln_g'][None, :], p['ln_b'][None, :]
        if li + 1 < len(layers):
            wn, bn_ = cat_w(layers[li + 1])
            h, k, q, v, s = _combine_ln_proj(agg2, s, ln_g, ln_b,
                                             wn.T, bn_[None, :])
            hs.append(h)
        else:
            return _combine_ln_final(agg2, s, ln_g, ln_b, hs[0], hs[1],
                                     params['W_hidden'].T,
                                     params['b_hidden'][None, :])
